# baseline (speedup 1.0000x reference)
import functools

import jax
import jax.numpy as jnp
from jax import lax
from jax.experimental import pallas as pl
from jax.experimental.pallas import tpu as pltpu
from jax.experimental.pallas import tpu_sc as plsc

_T = 2048
_D = 768
_H = 2048
_E = 8
_K = 2
_R = 256
_NB = (_T * _K + _E * _R) // _R
_N = _NB * _R
_NEG = -1e30


def _lane_cumsum(x):
    n = x.shape[-1]
    s = 1
    while s < n:
        shifted = jnp.concatenate(
            [jnp.zeros(x.shape[:-1] + (s,), x.dtype), x[..., : n - s]], axis=-1)
        x = x + shifted
        s *= 2
    return x


def _sublane_cumsum8(x):
    for s in (1, 2, 4):
        x = x + jnp.concatenate(
            [jnp.zeros((s, 1), x.dtype), x[: 8 - s]], axis=0)
    return x


def _router_body(flat_ref, rw_ref, pos_ref, wts_ref, be_ref, nb_ref):
    logits = lax.dot_general(rw_ref[...], flat_ref[...],
                             (((1,), (1,)), ((), ())),
                             preferred_element_type=jnp.float32)
    erange = lax.broadcasted_iota(jnp.int32, (_E, _T), 0)
    i1 = jnp.argmax(logits, axis=0).astype(jnp.int32)
    oh1 = (erange == i1[None, :])
    v1 = jnp.max(logits, axis=0)
    masked = jnp.where(oh1, _NEG, logits)
    i2 = jnp.argmax(masked, axis=0).astype(jnp.int32)
    oh2 = (erange == i2[None, :])
    v2 = jnp.max(masked, axis=0)
    g2 = jnp.exp(v2 - v1)
    w1 = 1.0 / (1.0 + g2)
    wts_ref[0, :] = w1
    wts_ref[1, :] = g2 * w1

    c1 = _lane_cumsum(oh1.astype(jnp.int32))
    c2 = _lane_cumsum(oh2.astype(jnp.int32))
    tot1 = c1[:, _T - 1:_T]
    sizes = tot1 + c2[:, _T - 1:_T]
    padded = ((sizes + (_R - 1)) // _R) * _R
    bounds = _sublane_cumsum8(padded)
    base = bounds - padded
    oh1i = oh1.astype(jnp.int32)
    oh2i = oh2.astype(jnp.int32)
    pos_ref[0, :] = jnp.sum(oh1i * (c1 + base), axis=0) - 1
    pos_ref[1, :] = jnp.sum(oh2i * (c2 + tot1 + base), axis=0) - 1

    nbb = bounds // _R
    brange = lax.broadcasted_iota(jnp.int32, (_E, _NB), 1)
    be = jnp.sum((nbb <= brange).astype(jnp.int32), axis=0)
    be_ref[0, :] = jnp.minimum(be, _E - 1)
    nb_ref[...] = nbb[_E - 1:_E, :]


def _router(flat, router_w):
    return pl.pallas_call(
        _router_body,
        out_shape=(
            jax.ShapeDtypeStruct((_K, _T), jnp.int32),
            jax.ShapeDtypeStruct((_K, _T), jnp.float32),
            jax.ShapeDtypeStruct((1, _NB), jnp.int32),
            jax.ShapeDtypeStruct((1, 1), jnp.int32),
        ),
    )(flat, router_w)


def _dispatch(flat, pos):
    info = plsc.get_sparse_core_info()
    nw = info.num_cores * info.num_subcores
    chunk = (_T * _K) // nw
    mesh = plsc.VectorSubcoreMesh(core_axis_name="c", subcore_axis_name="s")

    @functools.partial(
        pl.kernel, mesh=mesh,
        out_type=jax.ShapeDtypeStruct((_N, _D), jnp.float32),
        scratch_types=[
            pltpu.VMEM((chunk,), jnp.int32),
            pltpu.VMEM((chunk, _D), jnp.float32),
            pltpu.SemaphoreType.DMA,
        ],
    )
    def k(flat_hbm, pos_hbm, xs_hbm, idx_v, rows_v, sem):
        wid = lax.axis_index("s") * info.num_cores + lax.axis_index("c")
        base = wid * chunk
        slot = base // _T
        tok = base - slot * _T
        pltpu.sync_copy(pos_hbm.at[slot, pl.ds(tok, chunk)], idx_v)
        pltpu.sync_copy(flat_hbm.at[pl.ds(tok, chunk), :], rows_v)
        pltpu.async_copy(rows_v, xs_hbm.at[idx_v], sem).wait()

    return k(flat, pos)


def _ffn_block(be_ref, nb_ref, xs_ref, w1_ref, w3_ref, w2_ref, ys_ref):
    b = pl.program_id(0)

    @pl.when(b < nb_ref[0, 0])
    def _():
        x = xs_ref[...]
        h1 = lax.dot_general(x, w1_ref[0], (((1,), (1,)), ((), ())),
                             preferred_element_type=jnp.float32)
        h3 = lax.dot_general(x, w3_ref[0], (((1,), (1,)), ((), ())),
                             preferred_element_type=jnp.float32)
        act = h1 * jax.nn.sigmoid(h1) * h3
        y = lax.dot_general(act, w2_ref[0], (((1,), (1,)), ((), ())),
                            preferred_element_type=jnp.float32)
        ys_ref[...] = y.astype(jnp.bfloat16)


def _grouped_ffn(block_expert, nb_used, xs, w1, w3, w2):
    grid_spec = pltpu.PrefetchScalarGridSpec(
        num_scalar_prefetch=2,
        grid=(_NB,),
        in_specs=[
            pl.BlockSpec((_R, _D), lambda b, be, nb: (b, 0)),
            pl.BlockSpec((1, _H, _D), lambda b, be, nb: (be[0, b], 0, 0)),
            pl.BlockSpec((1, _H, _D), lambda b, be, nb: (be[0, b], 0, 0)),
            pl.BlockSpec((1, _D, _H), lambda b, be, nb: (be[0, b], 0, 0)),
        ],
        out_specs=pl.BlockSpec((_R, _D), lambda b, be, nb: (b, 0)),
    )
    return pl.pallas_call(
        _ffn_block,
        grid_spec=grid_spec,
        out_shape=jax.ShapeDtypeStruct((_N, _D), jnp.bfloat16),
    )(block_expert, nb_used, xs, w1, w3, w2)


def kernel(x, router_w, w1, w2, w3):
    flat = x.reshape(_T, _D)
    pos, wts, block_expert, nb_used = _router(flat, router_w)
    xs = _dispatch(flat, pos)
    ys = _grouped_ffn(block_expert, nb_used, xs, w1, w3, w2)
    out = jnp.sum(wts[:, :, None] * ys[pos].astype(jnp.float32), axis=0)
    return out.reshape(x.shape)


# speedup vs baseline: 1.1212x; 1.1212x over previous
import functools

import jax
import jax.numpy as jnp
from jax import lax
from jax.experimental import pallas as pl
from jax.experimental.pallas import tpu as pltpu
from jax.experimental.pallas import tpu_sc as plsc

_T = 2048
_D = 768
_H = 2048
_E = 8
_K = 2
_R = 256
_NB = (_T * _K + _E * _R) // _R
_N = _NB * _R
_NEG = -1e30


def _lane_cumsum(x):
    n = x.shape[-1]
    s = 1
    while s < n:
        shifted = jnp.concatenate(
            [jnp.zeros(x.shape[:-1] + (s,), x.dtype), x[..., : n - s]], axis=-1)
        x = x + shifted
        s *= 2
    return x


def _sublane_cumsum8(x):
    for s in (1, 2, 4):
        x = x + jnp.concatenate(
            [jnp.zeros((s, 1), x.dtype), x[: 8 - s]], axis=0)
    return x


def _router_body(flat_ref, rw_ref, pos_ref, wts_ref, be_ref, nb_ref):
    logits = lax.dot_general(rw_ref[...], flat_ref[...],
                             (((1,), (1,)), ((), ())),
                             preferred_element_type=jnp.float32)
    erange = lax.broadcasted_iota(jnp.int32, (_E, _T), 0)
    i1 = jnp.argmax(logits, axis=0).astype(jnp.int32)
    oh1 = (erange == i1[None, :])
    v1 = jnp.max(logits, axis=0)
    masked = jnp.where(oh1, _NEG, logits)
    i2 = jnp.argmax(masked, axis=0).astype(jnp.int32)
    oh2 = (erange == i2[None, :])
    v2 = jnp.max(masked, axis=0)
    g2 = jnp.exp(v2 - v1)
    w1 = 1.0 / (1.0 + g2)
    wts_ref[0, :] = w1
    wts_ref[1, :] = g2 * w1

    c1 = _lane_cumsum(oh1.astype(jnp.int32))
    c2 = _lane_cumsum(oh2.astype(jnp.int32))
    tot1 = c1[:, _T - 1:_T]
    sizes = tot1 + c2[:, _T - 1:_T]
    padded = ((sizes + (_R - 1)) // _R) * _R
    bounds = _sublane_cumsum8(padded)
    base = bounds - padded
    oh1i = oh1.astype(jnp.int32)
    oh2i = oh2.astype(jnp.int32)
    pos_ref[0, :] = jnp.sum(oh1i * (c1 + base), axis=0) - 1
    pos_ref[1, :] = jnp.sum(oh2i * (c2 + tot1 + base), axis=0) - 1

    nbb = bounds // _R
    brange = lax.broadcasted_iota(jnp.int32, (_E, _NB), 1)
    be = jnp.sum((nbb <= brange).astype(jnp.int32), axis=0)
    be_ref[0, :] = jnp.minimum(be, _E - 1)
    nb_ref[...] = nbb[_E - 1:_E, :]


def _router(flat, router_w):
    return pl.pallas_call(
        _router_body,
        out_shape=(
            jax.ShapeDtypeStruct((_K, _T), jnp.int32),
            jax.ShapeDtypeStruct((_K, _T), jnp.float32),
            jax.ShapeDtypeStruct((1, _NB), jnp.int32),
            jax.ShapeDtypeStruct((1, 1), jnp.int32),
        ),
    )(flat, router_w)


def _dispatch(flat, pos):
    info = plsc.get_sparse_core_info()
    nw = info.num_cores * info.num_subcores
    chunk = (_T * _K) // nw
    mesh = plsc.VectorSubcoreMesh(core_axis_name="c", subcore_axis_name="s")

    @functools.partial(
        pl.kernel, mesh=mesh,
        out_type=jax.ShapeDtypeStruct((_N, _D), jnp.float32),
        scratch_types=[
            pltpu.VMEM((chunk,), jnp.int32),
            pltpu.VMEM((chunk, _D), jnp.float32),
            pltpu.SemaphoreType.DMA,
        ],
    )
    def k(flat_hbm, pos_hbm, xs_hbm, idx_v, rows_v, sem):
        wid = lax.axis_index("s") * info.num_cores + lax.axis_index("c")
        base = wid * chunk
        slot = base // _T
        tok = base - slot * _T
        pltpu.sync_copy(pos_hbm.at[slot, pl.ds(tok, chunk)], idx_v)
        pltpu.sync_copy(flat_hbm.at[pl.ds(tok, chunk), :], rows_v)
        pltpu.async_copy(rows_v, xs_hbm.at[idx_v], sem).wait()

    return k(flat, pos)


def _ffn_block(be_ref, nb_ref, xs_ref, w1_ref, w3_ref, w2_ref, ys_ref):
    b = pl.program_id(0)

    @pl.when(b < nb_ref[0, 0])
    def _():
        x = xs_ref[...]
        h1 = lax.dot_general(x, w1_ref[0], (((1,), (1,)), ((), ())),
                             preferred_element_type=jnp.float32)
        h3 = lax.dot_general(x, w3_ref[0], (((1,), (1,)), ((), ())),
                             preferred_element_type=jnp.float32)
        act = h1 * jax.nn.sigmoid(h1) * h3
        ys_ref[...] = lax.dot_general(act, w2_ref[0], (((1,), (1,)), ((), ())),
                                      preferred_element_type=jnp.float32)


def _grouped_ffn(block_expert, nb_used, xs, w1, w3, w2):
    grid_spec = pltpu.PrefetchScalarGridSpec(
        num_scalar_prefetch=2,
        grid=(_NB,),
        in_specs=[
            pl.BlockSpec((_R, _D), lambda b, be, nb: (b, 0)),
            pl.BlockSpec((1, _H, _D), lambda b, be, nb: (be[0, b], 0, 0)),
            pl.BlockSpec((1, _H, _D), lambda b, be, nb: (be[0, b], 0, 0)),
            pl.BlockSpec((1, _D, _H), lambda b, be, nb: (be[0, b], 0, 0)),
        ],
        out_specs=pl.BlockSpec((_R, _D), lambda b, be, nb: (b, 0)),
    )
    return pl.pallas_call(
        _ffn_block,
        grid_spec=grid_spec,
        out_shape=jax.ShapeDtypeStruct((_N, _D), jnp.float32),
    )(block_expert, nb_used, xs, w1, w3, w2)


def kernel(x, router_w, w1, w2, w3):
    flat = x.reshape(_T, _D)
    pos, wts, block_expert, nb_used = _router(flat, router_w)
    xs = _dispatch(flat, pos)
    ys = _grouped_ffn(block_expert, nb_used, xs, w1, w3, w2)
    out = jnp.sum(wts[:, :, None] * ys[pos], axis=0)
    return out.reshape(x.shape)
